# baseline (speedup 1.0000x reference)
import functools

import jax
import jax.numpy as jnp
from jax import lax
from jax.experimental import pallas as pl
from jax.experimental.pallas import tpu as pltpu
from jax.experimental.pallas import tpu_sc as plsc

B = 4
S = 8192
D = 2048
LANES = 16
N_CORES = 2
BPC = B // N_CORES
SPB = 16 // BPC
CHUNK = S // SPB
STEPS = CHUNK // LANES
COLS = D // SPB

_mesh = plsc.VectorSubcoreMesh(
    core_axis_name="c", subcore_axis_name="s", num_cores=N_CORES)


@functools.partial(
    pl.kernel,
    mesh=_mesh,
    out_type=jax.ShapeDtypeStruct((B * D,), jnp.float32),
    scratch_types=[
        pltpu.VMEM((CHUNK,), jnp.int32),
        pltpu.VMEM((LANES,), jnp.int32),
        pltpu.VMEM_SHARED((16 * LANES,), jnp.int32),
        pltpu.VMEM((SPB * LANES,), jnp.int32),
        pltpu.VMEM((LANES,), jnp.int32),
        pltpu.VMEM((1, COLS), jnp.float32),
        pltpu.VMEM((COLS,), jnp.float32),
        pltpu.SemaphoreType.DMA,
        pltpu.SemaphoreType.DMA,
        pltpu.SemaphoreType.DMA,
    ],
)
def _clip_argmax_sc(hidden_hbm, ids_hbm, out_hbm,
                    ids_v, acc_v, shared_keys, part_v, idx_v, rows_v,
                    out_v, sem, sem2, sem3):
    c = lax.axis_index("c")
    s = lax.axis_index("s")
    b = c * BPC + s // SPB
    bc = s // SPB
    chunk = s % SPB
    base = chunk * CHUNK
    lane = lax.iota(jnp.int32, LANES)

    HALF = CHUNK // 2
    cp1 = pltpu.async_copy(
        ids_hbm.at[pl.ds(b * S + base, HALF)], ids_v.at[pl.ds(0, HALF)], sem)
    cp2 = pltpu.async_copy(
        ids_hbm.at[pl.ds(b * S + base + HALF, HALF)],
        ids_v.at[pl.ds(HALF, HALF)], sem2)

    cvec0 = (S - 1 - base) - lane

    def step(j, carry):
        acc, cvec = carry
        v = ids_v[pl.ds(j * LANES, LANES)]
        key = (v << 13) + cvec
        return jnp.maximum(acc, key), cvec - LANES

    init = (jnp.full((LANES,), -2**31, jnp.int32), cvec0)
    cp1.wait()
    half1 = lax.fori_loop(0, STEPS // 2, step, init, unroll=8)
    cp2.wait()
    acc, _ = lax.fori_loop(STEPS // 2, STEPS, step, half1, unroll=8)
    acc_v[...] = acc
    pltpu.sync_copy(acc_v, shared_keys.at[pl.ds(s * LANES, LANES)])
    plsc.subcore_barrier()

    pltpu.sync_copy(
        shared_keys.at[pl.ds(bc * SPB * LANES, SPB * LANES)], part_v)
    m = part_v[pl.ds(0, LANES)]
    for i in range(1, SPB):
        m = jnp.maximum(m, part_v[pl.ds(i * LANES, LANES)])
    best = m[0]
    for i in range(1, LANES):
        best = jnp.maximum(best, m[i])
    idx = (S - 1) - lax.rem(best, S)

    idx_v[...] = jnp.full((LANES,), 0, jnp.int32) + (b * S + idx)
    pltpu.async_copy(
        hidden_hbm.at[idx_v.at[pl.ds(0, 1)], pl.ds(chunk * COLS, COLS)],
        rows_v, sem).wait()

    HCOL = COLS // 2
    for j in range(HCOL // LANES):
        x = rows_v[0, pl.ds(j * LANES, LANES)]
        x2 = x * x
        out_v[pl.ds(j * LANES, LANES)] = x2 * x2
    st1 = pltpu.async_copy(
        out_v.at[pl.ds(0, HCOL)],
        out_hbm.at[pl.ds(b * D + chunk * COLS, HCOL)], sem3)
    for j in range(HCOL // LANES, COLS // LANES):
        x = rows_v[0, pl.ds(j * LANES, LANES)]
        x2 = x * x
        out_v[pl.ds(j * LANES, LANES)] = x2 * x2
    st2 = pltpu.async_copy(
        out_v.at[pl.ds(HCOL, HCOL)],
        out_hbm.at[pl.ds(b * D + chunk * COLS + HCOL, HCOL)], sem)
    st1.wait()
    st2.wait()


def kernel(last_hidden_state, input_ids):
    ids = input_ids.astype(jnp.int32).reshape(B * S)
    hidden = last_hidden_state.reshape(B * S, D)
    return _clip_argmax_sc(hidden, ids).reshape(B, D)


# speedup vs baseline: 1.0477x; 1.0477x over previous
import functools

import jax
import jax.numpy as jnp
from jax import lax
from jax.experimental import pallas as pl
from jax.experimental.pallas import tpu as pltpu
from jax.experimental.pallas import tpu_sc as plsc

B = 4
S = 8192
D = 2048
LANES = 16
N_CORES = 1
BPC = B // N_CORES
SPB = 16 // BPC
CHUNK = S // SPB
STEPS = CHUNK // LANES
COLS = D // SPB

_mesh = plsc.VectorSubcoreMesh(
    core_axis_name="c", subcore_axis_name="s", num_cores=N_CORES)


@functools.partial(
    pl.kernel,
    mesh=_mesh,
    out_type=jax.ShapeDtypeStruct((B * D,), jnp.float32),
    scratch_types=[
        pltpu.VMEM((CHUNK,), jnp.int32),
        pltpu.VMEM((LANES,), jnp.int32),
        pltpu.VMEM_SHARED((16 * LANES,), jnp.int32),
        pltpu.VMEM((SPB * LANES,), jnp.int32),
        pltpu.VMEM((LANES,), jnp.int32),
        pltpu.VMEM((1, COLS), jnp.float32),
        pltpu.VMEM((COLS,), jnp.float32),
        pltpu.SemaphoreType.DMA,
        pltpu.SemaphoreType.DMA,
        pltpu.SemaphoreType.DMA,
    ],
)
def _clip_argmax_sc(hidden_hbm, ids_hbm, out_hbm,
                    ids_v, acc_v, shared_keys, part_v, idx_v, rows_v,
                    out_v, sem, sem2, sem3):
    c = lax.axis_index("c")
    s = lax.axis_index("s")
    b = c * BPC + s // SPB
    bc = s // SPB
    chunk = s % SPB
    base = chunk * CHUNK
    lane = lax.iota(jnp.int32, LANES)

    HALF = CHUNK // 2
    cp1 = pltpu.async_copy(
        ids_hbm.at[pl.ds(b * S + base, HALF)], ids_v.at[pl.ds(0, HALF)], sem)
    cp2 = pltpu.async_copy(
        ids_hbm.at[pl.ds(b * S + base + HALF, HALF)],
        ids_v.at[pl.ds(HALF, HALF)], sem2)

    cvec0 = (S - 1 - base) - lane

    def step(j, carry):
        acc, cvec = carry
        v = ids_v[pl.ds(j * LANES, LANES)]
        key = (v << 13) + cvec
        return jnp.maximum(acc, key), cvec - LANES

    init = (jnp.full((LANES,), -2**31, jnp.int32), cvec0)
    cp1.wait()
    half1 = lax.fori_loop(0, STEPS // 2, step, init, unroll=16)
    cp2.wait()
    acc, _ = lax.fori_loop(STEPS // 2, STEPS, step, half1, unroll=16)
    acc_v[...] = acc
    pltpu.sync_copy(acc_v, shared_keys.at[pl.ds(s * LANES, LANES)])
    plsc.subcore_barrier()

    pltpu.sync_copy(
        shared_keys.at[pl.ds(bc * SPB * LANES, SPB * LANES)], part_v)
    m = part_v[pl.ds(0, LANES)]
    for i in range(1, SPB):
        m = jnp.maximum(m, part_v[pl.ds(i * LANES, LANES)])
    best = m[0]
    for i in range(1, LANES):
        best = jnp.maximum(best, m[i])
    idx = (S - 1) - lax.rem(best, S)

    idx_v[...] = jnp.full((LANES,), 0, jnp.int32) + (b * S + idx)
    pltpu.async_copy(
        hidden_hbm.at[idx_v.at[pl.ds(0, 1)], pl.ds(chunk * COLS, COLS)],
        rows_v, sem).wait()

    HCOL = COLS // 2
    for j in range(HCOL // LANES):
        x = rows_v[0, pl.ds(j * LANES, LANES)]
        x2 = x * x
        out_v[pl.ds(j * LANES, LANES)] = x2 * x2
    st1 = pltpu.async_copy(
        out_v.at[pl.ds(0, HCOL)],
        out_hbm.at[pl.ds(b * D + chunk * COLS, HCOL)], sem3)
    for j in range(HCOL // LANES, COLS // LANES):
        x = rows_v[0, pl.ds(j * LANES, LANES)]
        x2 = x * x
        out_v[pl.ds(j * LANES, LANES)] = x2 * x2
    st2 = pltpu.async_copy(
        out_v.at[pl.ds(HCOL, HCOL)],
        out_hbm.at[pl.ds(b * D + chunk * COLS + HCOL, HCOL)], sem)
    st1.wait()
    st2.wait()


def kernel(last_hidden_state, input_ids):
    ids = input_ids.astype(jnp.int32).reshape(B * S)
    hidden = last_hidden_state.reshape(B * S, D)
    return _clip_argmax_sc(hidden, ids).reshape(B, D)


# speedup vs baseline: 1.0610x; 1.0127x over previous
import functools

import jax
import jax.numpy as jnp
from jax import lax
from jax.experimental import pallas as pl
from jax.experimental.pallas import tpu as pltpu
from jax.experimental.pallas import tpu_sc as plsc

B = 4
S = 8192
D = 2048
LANES = 16
N_CORES = 1
BPC = B // N_CORES
SPB = 16 // BPC
CHUNK = S // SPB
STEPS = CHUNK // LANES
COLS = D // SPB

_mesh = plsc.VectorSubcoreMesh(
    core_axis_name="c", subcore_axis_name="s", num_cores=N_CORES)


@functools.partial(
    pl.kernel,
    mesh=_mesh,
    out_type=jax.ShapeDtypeStruct((B * D,), jnp.float32),
    scratch_types=[
        pltpu.VMEM((CHUNK,), jnp.int32),
        pltpu.VMEM((LANES,), jnp.int32),
        pltpu.VMEM_SHARED((16 * LANES,), jnp.int32),
        pltpu.VMEM((SPB * LANES,), jnp.int32),
        pltpu.VMEM((LANES,), jnp.int32),
        pltpu.VMEM((1, COLS), jnp.float32),
        pltpu.VMEM((COLS,), jnp.float32),
        pltpu.SemaphoreType.DMA,
        pltpu.SemaphoreType.DMA,
        pltpu.SemaphoreType.DMA,
    ],
)
def _clip_argmax_sc(hidden_hbm, ids_hbm, out_hbm,
                    ids_v, acc_v, shared_keys, part_v, idx_v, rows_v,
                    out_v, sem, sem2, sem3):
    c = lax.axis_index("c")
    s = lax.axis_index("s")
    b = c * BPC + s // SPB
    bc = s // SPB
    chunk = s % SPB
    base = chunk * CHUNK
    lane = lax.iota(jnp.int32, LANES)

    HALF = CHUNK // 2
    cp1 = pltpu.async_copy(
        ids_hbm.at[pl.ds(b * S + base, HALF)], ids_v.at[pl.ds(0, HALF)], sem)
    cp2 = pltpu.async_copy(
        ids_hbm.at[pl.ds(b * S + base + HALF, HALF)],
        ids_v.at[pl.ds(HALF, HALF)], sem2)

    cvec0 = (S - 1 - base) - lane

    def step(j, carry):
        acc, cvec = carry
        v = ids_v[pl.ds(j * LANES, LANES)]
        key = (v << 13) + cvec
        return jnp.maximum(acc, key), cvec - LANES

    init = (jnp.full((LANES,), -2**31, jnp.int32), cvec0)
    cp1.wait()
    half1 = lax.fori_loop(0, STEPS // 2, step, init, unroll=8)
    cp2.wait()
    acc, _ = lax.fori_loop(STEPS // 2, STEPS, step, half1, unroll=8)
    acc_v[...] = acc
    pltpu.sync_copy(acc_v, shared_keys.at[pl.ds(s * LANES, LANES)])
    plsc.subcore_barrier()

    pltpu.sync_copy(
        shared_keys.at[pl.ds(bc * SPB * LANES, SPB * LANES)], part_v)
    m = part_v[pl.ds(0, LANES)]
    for i in range(1, SPB):
        m = jnp.maximum(m, part_v[pl.ds(i * LANES, LANES)])
    best = m[0]
    for i in range(1, LANES):
        best = jnp.maximum(best, m[i])
    idx = (S - 1) - lax.rem(best, S)

    idx_v[...] = jnp.full((LANES,), 0, jnp.int32) + (b * S + idx)
    pltpu.async_copy(
        hidden_hbm.at[idx_v.at[pl.ds(0, 1)], pl.ds(chunk * COLS, COLS)],
        rows_v, sem).wait()

    HCOL = COLS // 2
    for j in range(HCOL // LANES):
        x = rows_v[0, pl.ds(j * LANES, LANES)]
        x2 = x * x
        out_v[pl.ds(j * LANES, LANES)] = x2 * x2
    st1 = pltpu.async_copy(
        out_v.at[pl.ds(0, HCOL)],
        out_hbm.at[pl.ds(b * D + chunk * COLS, HCOL)], sem3)
    for j in range(HCOL // LANES, COLS // LANES):
        x = rows_v[0, pl.ds(j * LANES, LANES)]
        x2 = x * x
        out_v[pl.ds(j * LANES, LANES)] = x2 * x2
    st2 = pltpu.async_copy(
        out_v.at[pl.ds(HCOL, HCOL)],
        out_hbm.at[pl.ds(b * D + chunk * COLS + HCOL, HCOL)], sem)
    st1.wait()
    st2.wait()


def kernel(last_hidden_state, input_ids):
    ids = input_ids.astype(jnp.int32).reshape(B * S)
    hidden = last_hidden_state.reshape(B * S, D)
    return _clip_argmax_sc(hidden, ids).reshape(B, D)
